# Initial kernel scaffold; baseline (speedup 1.0000x reference)
import jax
import jax.numpy as jnp
from jax.experimental import pallas as pl


def kernel(h, edge_index, W1, b1, W2, b2, W3, b3, W4, b4, Wl1, bl1, Wl2, bl2, Wo, bo):
    raise NotImplementedError("write your pallas kernel here")


# speedup vs baseline: 4.5707x; 4.5707x over previous
import functools

import jax
import jax.numpy as jnp
from jax import lax
from jax.experimental import pallas as pl
from jax.experimental.pallas import tpu as pltpu
from jax.experimental.pallas import tpu_sc as plsc

N = 10000
E = 320000
D = 128

NC = 2
NS = 16
NW = NC * NS
EPW = E // NW
CHUNK = 80
NCHUNK = EPW // CHUNK
TBASE = 632
ZCH = 80


def _for_tile_slices(s, fn):
    @pl.when(s < NS - 1)
    def _():
        for k in range(7):
            fn(s * TBASE + k * ZCH, ZCH)
        fn(s * TBASE + 7 * ZCH, 72)

    @pl.when(s == NS - 1)
    def _():
        for k in range(6):
            fn((NS - 1) * TBASE + k * ZCH, ZCH)
        fn((NS - 1) * TBASE + 6 * ZCH, 40)

def _sc_degree_body(src_hbm, dst_hbm, dout_hbm, din_hbm,
                    idx_v, ones_v, zbuf, acc_sh):
    c = lax.axis_index("c")
    s = lax.axis_index("s")
    w = c * NS + s
    ebase = w * EPW

    def fill_ones(i, carry):
        for j in range(D // 16):
            ones_v[i, pl.ds(j * 16, 16)] = jnp.full((16,), 1.0, jnp.float32)
        return carry

    lax.fori_loop(0, CHUNK, fill_ones, 0)

    def fill_zeros(i, carry):
        for j in range(D // 16):
            zbuf[i, pl.ds(j * 16, 16)] = jnp.zeros((16,), jnp.float32)
        return carry

    lax.fori_loop(0, ZCH, fill_zeros, 0)

    def zero_piece(off, nr):
        pltpu.sync_copy(zbuf.at[pl.ds(0, nr)], acc_sh.at[pl.ds(off, nr)])

    for idx_hbm, out_hbm in ((src_hbm, dout_hbm), (dst_hbm, din_hbm)):
        _for_tile_slices(s, zero_piece)
        plsc.subcore_barrier()

        def body(i, carry):
            base = ebase + i * CHUNK
            pltpu.sync_copy(idx_hbm.at[pl.ds(base, CHUNK)], idx_v)
            pltpu.sync_copy(ones_v, acc_sh.at[idx_v], add=True)
            return carry

        lax.fori_loop(0, NCHUNK, body, 0)
        plsc.subcore_barrier()

        def export_piece(off, nr):
            pltpu.sync_copy(acc_sh.at[pl.ds(off, nr)],
                            out_hbm.at[pl.ds(c * N + off, nr)])

        _for_tile_slices(s, export_piece)
        plsc.subcore_barrier()


def _sc_agg_body(xs_hbm, src_hbm, dst_hbm, part_hbm,
                 idx_s, idx_d, rows, zbuf, acc_sh, sem):
    c = lax.axis_index("c")
    s = lax.axis_index("s")
    w = c * NS + s

    def fill_zeros(i, carry):
        for j in range(D // 16):
            zbuf[i, pl.ds(j * 16, 16)] = jnp.zeros((16,), jnp.float32)
        return carry

    lax.fori_loop(0, ZCH, fill_zeros, 0)

    def zero_piece(off, nr):
        pltpu.sync_copy(zbuf.at[pl.ds(0, nr)], acc_sh.at[pl.ds(off, nr)])

    _for_tile_slices(s, zero_piece)
    plsc.subcore_barrier()

    ebase = w * EPW

    def body(i, carry):
        base = ebase + i * CHUNK
        pltpu.sync_copy(src_hbm.at[pl.ds(base, CHUNK)], idx_s)
        pltpu.sync_copy(dst_hbm.at[pl.ds(base, CHUNK)], idx_d)
        pltpu.async_copy(xs_hbm.at[idx_s], rows, sem).wait()
        pltpu.sync_copy(rows, acc_sh.at[idx_d], add=True)
        return carry

    lax.fori_loop(0, NCHUNK, body, 0)
    plsc.subcore_barrier()

    def export_piece(off, nr):
        pltpu.sync_copy(acc_sh.at[pl.ds(off, nr)],
                        part_hbm.at[pl.ds(c * N + off, nr)])

    _for_tile_slices(s, export_piece)


@functools.cache
def _build_sc_kernels():
    mesh = plsc.VectorSubcoreMesh(core_axis_name="c", subcore_axis_name="s",
                                  num_cores=NC, num_subcores=NS)
    sc_degree = pl.kernel(
        _sc_degree_body,
        out_type=[
            jax.ShapeDtypeStruct((NC * N, D), jnp.float32),
            jax.ShapeDtypeStruct((NC * N, D), jnp.float32),
        ],
        mesh=mesh,
        scratch_types=[
            pltpu.VMEM((CHUNK,), jnp.int32),
            pltpu.VMEM((CHUNK, D), jnp.float32),
            pltpu.VMEM((ZCH, D), jnp.float32),
            pltpu.VMEM_SHARED((N, D), jnp.float32),
        ],
    )
    sc_agg = pl.kernel(
        _sc_agg_body,
        out_type=jax.ShapeDtypeStruct((NC * N, D), jnp.float32),
        mesh=mesh,
        scratch_types=[
            pltpu.VMEM((CHUNK,), jnp.int32),
            pltpu.VMEM((CHUNK,), jnp.int32),
            pltpu.VMEM((CHUNK, D), jnp.float32),
            pltpu.VMEM((ZCH, D), jnp.float32),
            pltpu.VMEM_SHARED((N, D), jnp.float32),
            pltpu.SemaphoreType.DMA,
        ],
    )
    return sc_degree, sc_agg


_BLK = 1000
_G = N // _BLK


def _tc_prep_body(h_ref, do0, do1, di0, di1, invo_ref, invi_ref, xs_ref):
    deg_o = do0[...][:, 0:1] + do1[...][:, 0:1]
    deg_i = di0[...][:, 0:1] + di1[...][:, 0:1]
    invo = lax.rsqrt(jnp.maximum(deg_o, 1.0))
    invi = lax.rsqrt(jnp.maximum(deg_i, 1.0))
    invo_ref[...] = invo
    invi_ref[...] = invi
    xs_ref[...] = h_ref[...] * invo


def _tc_prep(h, dout, din):
    return pl.pallas_call(
        _tc_prep_body,
        grid=(_G,),
        in_specs=[
            pl.BlockSpec((_BLK, D), lambda i: (i, 0)),
            pl.BlockSpec((_BLK, D), lambda i: (i, 0)),
            pl.BlockSpec((_BLK, D), lambda i: (i + _G, 0)),
            pl.BlockSpec((_BLK, D), lambda i: (i, 0)),
            pl.BlockSpec((_BLK, D), lambda i: (i + _G, 0)),
        ],
        out_specs=[
            pl.BlockSpec((_BLK, 1), lambda i: (i, 0)),
            pl.BlockSpec((_BLK, 1), lambda i: (i, 0)),
            pl.BlockSpec((_BLK, D), lambda i: (i, 0)),
        ],
        out_shape=[
            jax.ShapeDtypeStruct((N, 1), jnp.float32),
            jax.ShapeDtypeStruct((N, 1), jnp.float32),
            jax.ShapeDtypeStruct((N, D), jnp.float32),
        ],
    )(h, dout, dout, din, din)


def _tc_layer_body(p0, p1, invi, invo, w_ref, b_ref, out_ref):
    m = (p0[...] + p1[...]) * invi[...]
    y = jnp.dot(m, w_ref[...], preferred_element_type=jnp.float32) + b_ref[...]
    out_ref[...] = jnp.maximum(y, 0.0) * invo[...]


def _tc_layer(part, invi, invo, w, b):
    return pl.pallas_call(
        _tc_layer_body,
        grid=(_G,),
        in_specs=[
            pl.BlockSpec((_BLK, D), lambda i: (i, 0)),
            pl.BlockSpec((_BLK, D), lambda i: (i + _G, 0)),
            pl.BlockSpec((_BLK, 1), lambda i: (i, 0)),
            pl.BlockSpec((_BLK, 1), lambda i: (i, 0)),
            pl.BlockSpec((D, D), lambda i: (0, 0)),
            pl.BlockSpec((1, D), lambda i: (0, 0)),
        ],
        out_specs=pl.BlockSpec((_BLK, D), lambda i: (i, 0)),
        out_shape=jax.ShapeDtypeStruct((N, D), jnp.float32),
    )(part, part, invi, invo, w, b)


def _tc_final_body(p0, p1, invi, w4, b4, wl1, bl1, wl2, bl2, wo, bo,
                   out_ref, acc):
    i = pl.program_id(0)

    @pl.when(i == 0)
    def _():
        acc[...] = jnp.zeros_like(acc)

    m = (p0[...] + p1[...]) * invi[...]
    y = jnp.dot(m, w4[...], preferred_element_type=jnp.float32) + b4[...]
    y = jnp.maximum(y, 0.0)
    acc[...] += jnp.sum(y, axis=0, keepdims=True)

    @pl.when(i == _G - 1)
    def _():
        hg = acc[...] * (1.0 / N)
        hg = jnp.dot(hg, wl1[...], preferred_element_type=jnp.float32) + bl1[...]
        hg = jnp.maximum(hg, 0.0)
        hg = jnp.dot(hg, wl2[...], preferred_element_type=jnp.float32) + bl2[...]
        hg = jnp.maximum(hg, 0.0)
        out_ref[...] = jnp.dot(hg, wo[...], preferred_element_type=jnp.float32) + bo[...]


def _tc_final(part, invi, w4, b4, wl1, bl1, wl2, bl2, wo, bo):
    return pl.pallas_call(
        _tc_final_body,
        grid=(_G,),
        in_specs=[
            pl.BlockSpec((_BLK, D), lambda i: (i, 0)),
            pl.BlockSpec((_BLK, D), lambda i: (i + _G, 0)),
            pl.BlockSpec((_BLK, 1), lambda i: (i, 0)),
            pl.BlockSpec((D, D), lambda i: (0, 0)),
            pl.BlockSpec((1, D), lambda i: (0, 0)),
            pl.BlockSpec((D, D), lambda i: (0, 0)),
            pl.BlockSpec((1, D), lambda i: (0, 0)),
            pl.BlockSpec((D, D), lambda i: (0, 0)),
            pl.BlockSpec((1, D), lambda i: (0, 0)),
            pl.BlockSpec((D, 1), lambda i: (0, 0)),
            pl.BlockSpec((1, 1), lambda i: (0, 0)),
        ],
        out_specs=pl.BlockSpec((1, 1), lambda i: (0, 0)),
        out_shape=jax.ShapeDtypeStruct((1, 1), jnp.float32),
        scratch_shapes=[pltpu.VMEM((1, D), jnp.float32)],
    )(part, part, invi, w4, b4, wl1, bl1, wl2, bl2, wo, bo)


def kernel(h, edge_index, W1, b1, W2, b2, W3, b3, W4, b4,
           Wl1, bl1, Wl2, bl2, Wo, bo):
    src = edge_index[0]
    dst = edge_index[1]

    _sc_degree, _sc_agg = _build_sc_kernels()
    dout, din = _sc_degree(src, dst)
    invo, invi, xs = _tc_prep(h, dout, din)

    for w, b in ((W1, b1), (W2, b2), (W3, b3)):
        part = _sc_agg(xs, src, dst)
        xs = _tc_layer(part, invi, invo, w, b.reshape(1, D))

    part = _sc_agg(xs, src, dst)
    return _tc_final(part, invi, W4, b4.reshape(1, D),
                     Wl1, bl1.reshape(1, D), Wl2, bl2.reshape(1, D),
                     Wo, bo.reshape(1, 1))


# speedup vs baseline: 9.7465x; 2.1324x over previous
import functools

import jax
import jax.numpy as jnp
from jax import lax
from jax.experimental import pallas as pl
from jax.experimental.pallas import tpu as pltpu
from jax.experimental.pallas import tpu_sc as plsc

N = 10000
E = 320000
D = 128

NC = 2
NS = 16
NW = NC * NS
EPW = E // NW
CHUNK = 125
NCHUNK = EPW // CHUNK
NPAIR = NCHUNK // 2
TBASE = 632
ZCH = 40


def _for_tile_slices(s, fn):
    @pl.when(s < NS - 1)
    def _():
        for k in range(15):
            fn(s * TBASE + k * ZCH, ZCH)
        fn(s * TBASE + 15 * ZCH, 32)

    @pl.when(s == NS - 1)
    def _():
        for k in range(13):
            fn((NS - 1) * TBASE + k * ZCH, ZCH)

def _sc_degree_body(cat_hbm, dout_hbm, din_hbm,
                    idx0, idx1, ones_v, zbuf, acc_sh, sem0, sem1):
    c = lax.axis_index("c")
    s = lax.axis_index("s")
    w = c * NS + s

    def fill_ones(i, carry):
        for j in range(D // 16):
            ones_v[i, pl.ds(j * 16, 16)] = jnp.full((16,), 1.0, jnp.float32)
        return carry

    lax.fori_loop(0, CHUNK, fill_ones, 0)

    def fill_zeros(i, carry):
        for j in range(D // 16):
            zbuf[i, pl.ds(j * 16, 16)] = jnp.zeros((16,), jnp.float32)
        return carry

    lax.fori_loop(0, ZCH, fill_zeros, 0)

    def zero_piece(off, nr):
        pltpu.sync_copy(zbuf.at[pl.ds(0, nr)], acc_sh.at[pl.ds(off, nr)])

    for p, out_hbm in ((0, dout_hbm), (1, din_hbm)):
        _for_tile_slices(s, zero_piece)
        plsc.subcore_barrier()

        def start_scatter(islot, sem):
            pltpu.async_copy(ones_v, acc_sh.at[islot.at[p]], sem, add=True)

        def wait_scatter(islot, sem):
            pltpu.make_async_copy(ones_v, acc_sh.at[islot.at[p]], sem).wait()

        pltpu.sync_copy(cat_hbm.at[w, 0], idx0)
        start_scatter(idx0, sem0)

        def body(j, carry):
            i0 = 2 * j
            i1 = i0 + 1
            pltpu.sync_copy(cat_hbm.at[w, i1], idx1)
            start_scatter(idx1, sem1)
            wait_scatter(idx0, sem0)

            @pl.when(j < NPAIR - 1)
            def _():
                pltpu.sync_copy(cat_hbm.at[w, i0 + 2], idx0)
                start_scatter(idx0, sem0)

            wait_scatter(idx1, sem1)
            return carry

        lax.fori_loop(0, NPAIR, body, 0)
        plsc.subcore_barrier()

        def export_piece(off, nr):
            pltpu.sync_copy(acc_sh.at[pl.ds(off, nr)],
                            out_hbm.at[pl.ds(c * N + off, nr)])

        _for_tile_slices(s, export_piece)
        plsc.subcore_barrier()


def _sc_agg_body(xs_hbm, cat_hbm, part_hbm,
                 idx0, idx1, rows0, rows1, zbuf, acc_sh, sem0, sem1):
    c = lax.axis_index("c")
    s = lax.axis_index("s")
    w = c * NS + s

    def fill_zeros(i, carry):
        for j in range(D // 16):
            zbuf[i, pl.ds(j * 16, 16)] = jnp.zeros((16,), jnp.float32)
        return carry

    lax.fori_loop(0, ZCH, fill_zeros, 0)

    def zero_piece(off, nr):
        pltpu.sync_copy(zbuf.at[pl.ds(0, nr)], acc_sh.at[pl.ds(off, nr)])

    _for_tile_slices(s, zero_piece)
    plsc.subcore_barrier()

    def start_gather(islot, rows, sem):
        pltpu.async_copy(xs_hbm.at[islot.at[0]], rows, sem)

    def wait_gather(islot, rows, sem):
        pltpu.make_async_copy(xs_hbm.at[islot.at[0]], rows, sem).wait()

    pltpu.sync_copy(cat_hbm.at[w, 0], idx0)
    start_gather(idx0, rows0, sem0)

    def body(j, carry):
        i0 = 2 * j
        i1 = i0 + 1
        pltpu.sync_copy(cat_hbm.at[w, i1], idx1)
        start_gather(idx1, rows1, sem1)
        wait_gather(idx0, rows0, sem0)
        pltpu.sync_copy(rows0, acc_sh.at[idx0.at[1]], add=True)

        @pl.when(j < NPAIR - 1)
        def _():
            pltpu.sync_copy(cat_hbm.at[w, i0 + 2], idx0)
            start_gather(idx0, rows0, sem0)

        wait_gather(idx1, rows1, sem1)
        pltpu.sync_copy(rows1, acc_sh.at[idx1.at[1]], add=True)
        return carry

    lax.fori_loop(0, NPAIR, body, 0)
    plsc.subcore_barrier()

    def export_piece(off, nr):
        pltpu.sync_copy(acc_sh.at[pl.ds(off, nr)],
                        part_hbm.at[pl.ds(c * N + off, nr)])

    _for_tile_slices(s, export_piece)


@functools.cache
def _build_sc_kernels():
    mesh = plsc.VectorSubcoreMesh(core_axis_name="c", subcore_axis_name="s",
                                  num_cores=NC, num_subcores=NS)
    sc_degree = pl.kernel(
        _sc_degree_body,
        out_type=[
            jax.ShapeDtypeStruct((NC * N, D), jnp.float32),
            jax.ShapeDtypeStruct((NC * N, D), jnp.float32),
        ],
        mesh=mesh,
        scratch_types=[
            pltpu.VMEM((2, CHUNK), jnp.int32),
            pltpu.VMEM((2, CHUNK), jnp.int32),
            pltpu.VMEM((CHUNK, D), jnp.float32),
            pltpu.VMEM((ZCH, D), jnp.float32),
            pltpu.VMEM_SHARED((N, D), jnp.float32),
            pltpu.SemaphoreType.DMA,
            pltpu.SemaphoreType.DMA,
        ],
    )
    sc_agg = pl.kernel(
        _sc_agg_body,
        out_type=jax.ShapeDtypeStruct((NC * N, D), jnp.float32),
        mesh=mesh,
        scratch_types=[
            pltpu.VMEM((2, CHUNK), jnp.int32),
            pltpu.VMEM((2, CHUNK), jnp.int32),
            pltpu.VMEM((CHUNK, D), jnp.float32),
            pltpu.VMEM((CHUNK, D), jnp.float32),
            pltpu.VMEM((ZCH, D), jnp.float32),
            pltpu.VMEM_SHARED((N, D), jnp.float32),
            pltpu.SemaphoreType.DMA,
            pltpu.SemaphoreType.DMA,
        ],
    )
    return sc_degree, sc_agg


_BLK = 1000
_G = N // _BLK


def _tc_prep_body(h_ref, do0, do1, di0, di1, invo_ref, invi_ref, xs_ref):
    deg_o = do0[...][:, 0:1] + do1[...][:, 0:1]
    deg_i = di0[...][:, 0:1] + di1[...][:, 0:1]
    invo = lax.rsqrt(jnp.maximum(deg_o, 1.0))
    invi = lax.rsqrt(jnp.maximum(deg_i, 1.0))
    invo_ref[...] = invo
    invi_ref[...] = invi
    xs_ref[...] = h_ref[...] * invo


def _tc_prep(h, dout, din):
    return pl.pallas_call(
        _tc_prep_body,
        grid=(_G,),
        in_specs=[
            pl.BlockSpec((_BLK, D), lambda i: (i, 0)),
            pl.BlockSpec((_BLK, D), lambda i: (i, 0)),
            pl.BlockSpec((_BLK, D), lambda i: (i + _G, 0)),
            pl.BlockSpec((_BLK, D), lambda i: (i, 0)),
            pl.BlockSpec((_BLK, D), lambda i: (i + _G, 0)),
        ],
        out_specs=[
            pl.BlockSpec((_BLK, 1), lambda i: (i, 0)),
            pl.BlockSpec((_BLK, 1), lambda i: (i, 0)),
            pl.BlockSpec((_BLK, D), lambda i: (i, 0)),
        ],
        out_shape=[
            jax.ShapeDtypeStruct((N, 1), jnp.float32),
            jax.ShapeDtypeStruct((N, 1), jnp.float32),
            jax.ShapeDtypeStruct((N, D), jnp.float32),
        ],
    )(h, dout, dout, din, din)


def _tc_layer_body(p0, p1, invi, invo, w_ref, b_ref, out_ref):
    m = (p0[...] + p1[...]) * invi[...]
    y = jnp.dot(m, w_ref[...], preferred_element_type=jnp.float32) + b_ref[...]
    out_ref[...] = jnp.maximum(y, 0.0) * invo[...]


def _tc_layer(part, invi, invo, w, b):
    return pl.pallas_call(
        _tc_layer_body,
        grid=(_G,),
        in_specs=[
            pl.BlockSpec((_BLK, D), lambda i: (i, 0)),
            pl.BlockSpec((_BLK, D), lambda i: (i + _G, 0)),
            pl.BlockSpec((_BLK, 1), lambda i: (i, 0)),
            pl.BlockSpec((_BLK, 1), lambda i: (i, 0)),
            pl.BlockSpec((D, D), lambda i: (0, 0)),
            pl.BlockSpec((1, D), lambda i: (0, 0)),
        ],
        out_specs=pl.BlockSpec((_BLK, D), lambda i: (i, 0)),
        out_shape=jax.ShapeDtypeStruct((N, D), jnp.float32),
    )(part, part, invi, invo, w, b)


def _tc_final_body(p0, p1, invi, w4, b4, wl1, bl1, wl2, bl2, wo, bo,
                   out_ref, acc):
    i = pl.program_id(0)

    @pl.when(i == 0)
    def _():
        acc[...] = jnp.zeros_like(acc)

    m = (p0[...] + p1[...]) * invi[...]
    y = jnp.dot(m, w4[...], preferred_element_type=jnp.float32) + b4[...]
    y = jnp.maximum(y, 0.0)
    acc[...] += jnp.sum(y, axis=0, keepdims=True)

    @pl.when(i == _G - 1)
    def _():
        hg = acc[...] * (1.0 / N)
        hg = jnp.dot(hg, wl1[...], preferred_element_type=jnp.float32) + bl1[...]
        hg = jnp.maximum(hg, 0.0)
        hg = jnp.dot(hg, wl2[...], preferred_element_type=jnp.float32) + bl2[...]
        hg = jnp.maximum(hg, 0.0)
        out_ref[...] = jnp.dot(hg, wo[...], preferred_element_type=jnp.float32) + bo[...]


def _tc_final(part, invi, w4, b4, wl1, bl1, wl2, bl2, wo, bo):
    return pl.pallas_call(
        _tc_final_body,
        grid=(_G,),
        in_specs=[
            pl.BlockSpec((_BLK, D), lambda i: (i, 0)),
            pl.BlockSpec((_BLK, D), lambda i: (i + _G, 0)),
            pl.BlockSpec((_BLK, 1), lambda i: (i, 0)),
            pl.BlockSpec((D, D), lambda i: (0, 0)),
            pl.BlockSpec((1, D), lambda i: (0, 0)),
            pl.BlockSpec((D, D), lambda i: (0, 0)),
            pl.BlockSpec((1, D), lambda i: (0, 0)),
            pl.BlockSpec((D, D), lambda i: (0, 0)),
            pl.BlockSpec((1, D), lambda i: (0, 0)),
            pl.BlockSpec((D, 1), lambda i: (0, 0)),
            pl.BlockSpec((1, 1), lambda i: (0, 0)),
        ],
        out_specs=pl.BlockSpec((1, 1), lambda i: (0, 0)),
        out_shape=jax.ShapeDtypeStruct((1, 1), jnp.float32),
        scratch_shapes=[pltpu.VMEM((1, D), jnp.float32)],
    )(part, part, invi, w4, b4, wl1, bl1, wl2, bl2, wo, bo)


def kernel(h, edge_index, W1, b1, W2, b2, W3, b3, W4, b4,
           Wl1, bl1, Wl2, bl2, Wo, bo):
    cat = jnp.transpose(edge_index.reshape(2, NW, NCHUNK, CHUNK), (1, 2, 0, 3))

    _sc_degree, _sc_agg = _build_sc_kernels()
    dout, din = _sc_degree(cat)
    invo, invi, xs = _tc_prep(h, dout, din)

    for w, b in ((W1, b1), (W2, b2), (W3, b3)):
        part = _sc_agg(xs, cat)
        xs = _tc_layer(part, invi, invo, w, b.reshape(1, D))

    part = _sc_agg(xs, cat)
    return _tc_final(part, invi, W4, b4.reshape(1, D),
                     Wl1, bl1.reshape(1, D), Wl2, bl2.reshape(1, D),
                     Wo, bo.reshape(1, 1))


# speedup vs baseline: 10.9499x; 1.1235x over previous
import functools

import jax
import jax.numpy as jnp
from jax import lax
from jax.experimental import pallas as pl
from jax.experimental.pallas import tpu as pltpu
from jax.experimental.pallas import tpu_sc as plsc

N = 10000
E = 320000
D = 128

NC = 2
NS = 16
NW = NC * NS
EPW = E // NW
CHUNK = 125
NCHUNK = EPW // CHUNK
NPAIR = NCHUNK // 2
G = 10
NGRP = NCHUNK // G
NGP = NGRP // 2
TBASE = 632
ZCH = 40


def _for_tile_slices(s, fn):
    @pl.when(s < NS - 1)
    def _():
        for k in range(15):
            fn(s * TBASE + k * ZCH, ZCH)
        fn(s * TBASE + 15 * ZCH, 32)

    @pl.when(s == NS - 1)
    def _():
        for k in range(13):
            fn((NS - 1) * TBASE + k * ZCH, ZCH)

def _sc_degree_body(cat_hbm, deg_hbm,
                    idx0, idx1, ones_s, ones_d, zbuf, acc_sh,
                    s0a, s0b, s1a, s1b):
    c = lax.axis_index("c")
    s = lax.axis_index("s")
    w = c * NS + s

    def fill_ones(i, carry):
        for j in range(D // 16):
            v1 = jnp.full((16,), 1.0, jnp.float32)
            v0 = jnp.zeros((16,), jnp.float32)
            ones_s[i, pl.ds(j * 16, 16)] = v1 if j == 0 else v0
            ones_d[i, pl.ds(j * 16, 16)] = v1 if j == 1 else v0
        return carry

    lax.fori_loop(0, CHUNK, fill_ones, 0)

    def fill_zeros(i, carry):
        for j in range(D // 16):
            zbuf[i, pl.ds(j * 16, 16)] = jnp.zeros((16,), jnp.float32)
        return carry

    lax.fori_loop(0, ZCH, fill_zeros, 0)

    def zero_piece(off, nr):
        pltpu.sync_copy(zbuf.at[pl.ds(0, nr)], acc_sh.at[pl.ds(off, nr)])

    _for_tile_slices(s, zero_piece)
    plsc.subcore_barrier()

    def start_scatter(islot, sa, sb):
        pltpu.async_copy(ones_s, acc_sh.at[islot.at[0]], sa, add=True)
        pltpu.async_copy(ones_d, acc_sh.at[islot.at[1]], sb, add=True)

    def wait_scatter(islot, sa, sb):
        pltpu.make_async_copy(ones_s, acc_sh.at[islot.at[0]], sa).wait()
        pltpu.make_async_copy(ones_d, acc_sh.at[islot.at[1]], sb).wait()

    pltpu.sync_copy(cat_hbm.at[w, 0], idx0)
    start_scatter(idx0, s0a, s0b)

    def body(j, carry):
        i0 = 2 * j
        i1 = i0 + 1
        pltpu.sync_copy(cat_hbm.at[w, i1], idx1)
        start_scatter(idx1, s1a, s1b)
        wait_scatter(idx0, s0a, s0b)

        @pl.when(j < NPAIR - 1)
        def _():
            pltpu.sync_copy(cat_hbm.at[w, i0 + 2], idx0)
            start_scatter(idx0, s0a, s0b)

        wait_scatter(idx1, s1a, s1b)
        return carry

    lax.fori_loop(0, NPAIR, body, 0)
    plsc.subcore_barrier()

    def export_piece(off, nr):
        pltpu.sync_copy(acc_sh.at[pl.ds(off, nr)],
                        deg_hbm.at[pl.ds(c * N + off, nr)])

    _for_tile_slices(s, export_piece)


def _sc_agg_body(xs_hbm, cat_hbm, part_hbm,
                 gidx0, gidx1, rows0, rows1, zbuf, acc_sh,
                 gsem0, gsem1, ssem0, ssem1, isem0, isem1):
    c = lax.axis_index("c")
    s = lax.axis_index("s")
    w = c * NS + s

    def fill_zeros(i, carry):
        for j in range(D // 16):
            zbuf[i, pl.ds(j * 16, 16)] = jnp.zeros((16,), jnp.float32)
        return carry

    lax.fori_loop(0, ZCH, fill_zeros, 0)

    def zero_piece(off, nr):
        pltpu.sync_copy(zbuf.at[pl.ds(0, nr)], acc_sh.at[pl.ds(off, nr)])

    _for_tile_slices(s, zero_piece)
    plsc.subcore_barrier()

    rows_b = (rows0, rows1)
    gsem_b = (gsem0, gsem1)
    ssem_b = (ssem0, ssem1)
    gslot_h = (gidx0, gidx1)
    isem_h = (isem0, isem1)

    def sgather(idx_ref, b):
        pltpu.async_copy(xs_hbm.at[idx_ref], rows_b[b], gsem_b[b])

    def wgather(b):
        pltpu.make_async_copy(xs_hbm.at[gidx0.at[0, 0]], rows_b[b],
                              gsem_b[b]).wait()

    def sscatter(idx_ref, b):
        pltpu.async_copy(rows_b[b], acc_sh.at[idx_ref], ssem_b[b], add=True)

    def wscatter(b):
        pltpu.make_async_copy(rows_b[b], acc_sh.at[gidx0.at[0, 1]],
                              ssem_b[b]).wait()

    def load_group(g, h, sem):
        pltpu.async_copy(cat_hbm.at[w, pl.ds(g * G, G)], gslot_h[h], sem)

    def wait_group(g, h, sem):
        pltpu.make_async_copy(cat_hbm.at[w, pl.ds(g * G, G)], gslot_h[h],
                              sem).wait()

    pltpu.sync_copy(cat_hbm.at[w, pl.ds(0, G)], gidx0)
    sgather(gidx0.at[0, 0], 0)

    def body(p, carry):
        for half in range(2):
            gslot = gslot_h[half]
            nslot = gslot_h[1 - half]

            for k_in in range(G):
                b = k_in % 2
                if half == 0 and k_in == 0:
                    @pl.when(p > 0)
                    def _():
                        wscatter(1 - b)
                else:
                    wscatter(1 - b)
                if k_in < G - 1:
                    sgather(gslot.at[k_in + 1, 0], 1 - b)
                else:
                    if half == 0:
                        wait_group(2 * p + 1, 1 - half, isem_h[1 - half])
                        sgather(nslot.at[0, 0], 1 - b)
                    else:
                        @pl.when(p < NGP - 1)
                        def _():
                            wait_group(2 * p + 2, 0, isem_h[0])
                            sgather(nslot.at[0, 0], 1 - b)
                wgather(b)
                sscatter(gslot.at[k_in, 1], b)
                if k_in == 0:
                    if half == 0:
                        load_group(2 * p + 1, 1, isem_h[1])
                    else:
                        @pl.when(p < NGP - 1)
                        def _():
                            load_group(2 * p + 2, 0, isem_h[0])
        return carry

    lax.fori_loop(0, NGP, body, 0)
    wscatter((NCHUNK - 1) % 2)
    plsc.subcore_barrier()

    def export_piece(off, nr):
        pltpu.sync_copy(acc_sh.at[pl.ds(off, nr)],
                        part_hbm.at[pl.ds(c * N + off, nr)])

    _for_tile_slices(s, export_piece)


@functools.cache
def _build_sc_kernels():
    mesh = plsc.VectorSubcoreMesh(core_axis_name="c", subcore_axis_name="s",
                                  num_cores=NC, num_subcores=NS)
    sc_degree = pl.kernel(
        _sc_degree_body,
        out_type=jax.ShapeDtypeStruct((NC * N, D), jnp.float32),
        mesh=mesh,
        scratch_types=[
            pltpu.VMEM((2, CHUNK), jnp.int32),
            pltpu.VMEM((2, CHUNK), jnp.int32),
            pltpu.VMEM((CHUNK, D), jnp.float32),
            pltpu.VMEM((CHUNK, D), jnp.float32),
            pltpu.VMEM((ZCH, D), jnp.float32),
            pltpu.VMEM_SHARED((N, D), jnp.float32),
            pltpu.SemaphoreType.DMA,
            pltpu.SemaphoreType.DMA,
            pltpu.SemaphoreType.DMA,
            pltpu.SemaphoreType.DMA,
        ],
    )
    sc_agg = pl.kernel(
        _sc_agg_body,
        out_type=jax.ShapeDtypeStruct((NC * N, D), jnp.float32),
        mesh=mesh,
        scratch_types=[
            pltpu.VMEM((G, 2, CHUNK), jnp.int32),
            pltpu.VMEM((G, 2, CHUNK), jnp.int32),
            pltpu.VMEM((CHUNK, D), jnp.float32),
            pltpu.VMEM((CHUNK, D), jnp.float32),
            pltpu.VMEM((ZCH, D), jnp.float32),
            pltpu.VMEM_SHARED((N, D), jnp.float32),
            pltpu.SemaphoreType.DMA,
            pltpu.SemaphoreType.DMA,
            pltpu.SemaphoreType.DMA,
            pltpu.SemaphoreType.DMA,
            pltpu.SemaphoreType.DMA,
            pltpu.SemaphoreType.DMA,
        ],
    )
    return sc_degree, sc_agg


_BLK = 1000
_G = N // _BLK


def _tc_prep_body(h_ref, d0, d1, invo_ref, invi_ref, xs_ref):
    dsum = d0[...] + d1[...]
    deg_o = dsum[:, 0:1]
    deg_i = dsum[:, 16:17]
    invo = lax.rsqrt(jnp.maximum(deg_o, 1.0))
    invi = lax.rsqrt(jnp.maximum(deg_i, 1.0))
    invo_ref[...] = invo
    invi_ref[...] = invi
    xs_ref[...] = h_ref[...] * invo


def _tc_prep(h, deg):
    return pl.pallas_call(
        _tc_prep_body,
        grid=(_G,),
        in_specs=[
            pl.BlockSpec((_BLK, D), lambda i: (i, 0)),
            pl.BlockSpec((_BLK, D), lambda i: (i, 0)),
            pl.BlockSpec((_BLK, D), lambda i: (i + _G, 0)),
        ],
        out_specs=[
            pl.BlockSpec((_BLK, 1), lambda i: (i, 0)),
            pl.BlockSpec((_BLK, 1), lambda i: (i, 0)),
            pl.BlockSpec((_BLK, D), lambda i: (i, 0)),
        ],
        out_shape=[
            jax.ShapeDtypeStruct((N, 1), jnp.float32),
            jax.ShapeDtypeStruct((N, 1), jnp.float32),
            jax.ShapeDtypeStruct((N, D), jnp.float32),
        ],
    )(h, deg, deg)


def _tc_layer_body(p0, p1, invi, invo, w_ref, b_ref, out_ref):
    m = (p0[...] + p1[...]) * invi[...]
    y = jnp.dot(m, w_ref[...], preferred_element_type=jnp.float32) + b_ref[...]
    out_ref[...] = jnp.maximum(y, 0.0) * invo[...]


def _tc_layer(part, invi, invo, w, b):
    return pl.pallas_call(
        _tc_layer_body,
        grid=(_G,),
        in_specs=[
            pl.BlockSpec((_BLK, D), lambda i: (i, 0)),
            pl.BlockSpec((_BLK, D), lambda i: (i + _G, 0)),
            pl.BlockSpec((_BLK, 1), lambda i: (i, 0)),
            pl.BlockSpec((_BLK, 1), lambda i: (i, 0)),
            pl.BlockSpec((D, D), lambda i: (0, 0)),
            pl.BlockSpec((1, D), lambda i: (0, 0)),
        ],
        out_specs=pl.BlockSpec((_BLK, D), lambda i: (i, 0)),
        out_shape=jax.ShapeDtypeStruct((N, D), jnp.float32),
    )(part, part, invi, invo, w, b)


def _tc_final_body(p0, p1, invi, w4, b4, wl1, bl1, wl2, bl2, wo, bo,
                   out_ref, acc):
    i = pl.program_id(0)

    @pl.when(i == 0)
    def _():
        acc[...] = jnp.zeros_like(acc)

    m = (p0[...] + p1[...]) * invi[...]
    y = jnp.dot(m, w4[...], preferred_element_type=jnp.float32) + b4[...]
    y = jnp.maximum(y, 0.0)
    acc[...] += jnp.sum(y, axis=0, keepdims=True)

    @pl.when(i == _G - 1)
    def _():
        hg = acc[...] * (1.0 / N)
        hg = jnp.dot(hg, wl1[...], preferred_element_type=jnp.float32) + bl1[...]
        hg = jnp.maximum(hg, 0.0)
        hg = jnp.dot(hg, wl2[...], preferred_element_type=jnp.float32) + bl2[...]
        hg = jnp.maximum(hg, 0.0)
        out_ref[...] = jnp.dot(hg, wo[...], preferred_element_type=jnp.float32) + bo[...]


def _tc_final(part, invi, w4, b4, wl1, bl1, wl2, bl2, wo, bo):
    return pl.pallas_call(
        _tc_final_body,
        grid=(_G,),
        in_specs=[
            pl.BlockSpec((_BLK, D), lambda i: (i, 0)),
            pl.BlockSpec((_BLK, D), lambda i: (i + _G, 0)),
            pl.BlockSpec((_BLK, 1), lambda i: (i, 0)),
            pl.BlockSpec((D, D), lambda i: (0, 0)),
            pl.BlockSpec((1, D), lambda i: (0, 0)),
            pl.BlockSpec((D, D), lambda i: (0, 0)),
            pl.BlockSpec((1, D), lambda i: (0, 0)),
            pl.BlockSpec((D, D), lambda i: (0, 0)),
            pl.BlockSpec((1, D), lambda i: (0, 0)),
            pl.BlockSpec((D, 1), lambda i: (0, 0)),
            pl.BlockSpec((1, 1), lambda i: (0, 0)),
        ],
        out_specs=pl.BlockSpec((1, 1), lambda i: (0, 0)),
        out_shape=jax.ShapeDtypeStruct((1, 1), jnp.float32),
        scratch_shapes=[pltpu.VMEM((1, D), jnp.float32)],
    )(part, part, invi, w4, b4, wl1, bl1, wl2, bl2, wo, bo)


def kernel(h, edge_index, W1, b1, W2, b2, W3, b3, W4, b4,
           Wl1, bl1, Wl2, bl2, Wo, bo):
    cat = jnp.transpose(edge_index.reshape(2, NW, NCHUNK, CHUNK), (1, 2, 0, 3))

    _sc_degree, _sc_agg = _build_sc_kernels()
    deg = _sc_degree(cat)
    invo, invi, xs = _tc_prep(h, deg)

    for w, b in ((W1, b1), (W2, b2), (W3, b3)):
        part = _sc_agg(xs, cat)
        xs = _tc_layer(part, invi, invo, w, b.reshape(1, D))

    part = _sc_agg(xs, cat)
    return _tc_final(part, invi, W4, b4.reshape(1, D),
                     Wl1, bl1.reshape(1, D), Wl2, bl2.reshape(1, D),
                     Wo, bo.reshape(1, 1))


# speedup vs baseline: 11.4070x; 1.0417x over previous
import functools

import jax
import jax.numpy as jnp
from jax import lax
from jax.experimental import pallas as pl
from jax.experimental.pallas import tpu as pltpu
from jax.experimental.pallas import tpu_sc as plsc

N = 10000
E = 320000
D = 128

NC = 2
NS = 16
NW = NC * NS
EPW = E // NW
CHUNK = 125
NCHUNK = EPW // CHUNK
NPAIR = NCHUNK // 2
G = 10
NGRP = NCHUNK // G
NGP = NGRP // 2
TBASE = 632
ZCH = 40


def _for_tile_slices(s, fn, wfn=None):
    @pl.when(s < NS - 1)
    def _():
        pieces = [(s * TBASE + k * ZCH, ZCH) for k in range(15)]
        pieces.append((s * TBASE + 15 * ZCH, 32))
        for off, nr in pieces:
            fn(off, nr)
        if wfn is not None:
            for off, nr in pieces:
                wfn(off, nr)

    @pl.when(s == NS - 1)
    def _():
        pieces = [((NS - 1) * TBASE + k * ZCH, ZCH) for k in range(13)]
        for off, nr in pieces:
            fn(off, nr)
        if wfn is not None:
            for off, nr in pieces:
                wfn(off, nr)

def _sc_degree_body(cat_hbm, deg_hbm,
                    idx0, idx1, ones_s, ones_d, zbuf, acc_sh,
                    s0a, s0b, s1a, s1b, psem):
    c = lax.axis_index("c")
    s = lax.axis_index("s")
    w = c * NS + s

    def fill_ones(i, carry):
        for j in range(D // 16):
            v1 = jnp.full((16,), 1.0, jnp.float32)
            v0 = jnp.zeros((16,), jnp.float32)
            ones_s[i, pl.ds(j * 16, 16)] = v1 if j == 0 else v0
            ones_d[i, pl.ds(j * 16, 16)] = v1 if j == 1 else v0
        return carry

    lax.fori_loop(0, CHUNK, fill_ones, 0)

    def fill_zeros(i, carry):
        for j in range(D // 16):
            zbuf[i, pl.ds(j * 16, 16)] = jnp.zeros((16,), jnp.float32)
        return carry

    lax.fori_loop(0, ZCH, fill_zeros, 0)

    def zero_start(off, nr):
        pltpu.async_copy(zbuf.at[pl.ds(0, nr)], acc_sh.at[pl.ds(off, nr)],
                         psem)

    def zero_wait(off, nr):
        pltpu.make_async_copy(zbuf.at[pl.ds(0, nr)],
                              acc_sh.at[pl.ds(off, nr)], psem).wait()

    _for_tile_slices(s, zero_start, zero_wait)
    plsc.subcore_barrier()

    def start_scatter(islot, sa, sb):
        pltpu.async_copy(ones_s, acc_sh.at[islot.at[0]], sa, add=True)
        pltpu.async_copy(ones_d, acc_sh.at[islot.at[1]], sb, add=True)

    def wait_scatter(islot, sa, sb):
        pltpu.make_async_copy(ones_s, acc_sh.at[islot.at[0]], sa).wait()
        pltpu.make_async_copy(ones_d, acc_sh.at[islot.at[1]], sb).wait()

    pltpu.sync_copy(cat_hbm.at[w, 0], idx0)
    start_scatter(idx0, s0a, s0b)

    def body(j, carry):
        i0 = 2 * j
        i1 = i0 + 1
        pltpu.sync_copy(cat_hbm.at[w, i1], idx1)
        start_scatter(idx1, s1a, s1b)
        wait_scatter(idx0, s0a, s0b)

        @pl.when(j < NPAIR - 1)
        def _():
            pltpu.sync_copy(cat_hbm.at[w, i0 + 2], idx0)
            start_scatter(idx0, s0a, s0b)

        wait_scatter(idx1, s1a, s1b)
        return carry

    lax.fori_loop(0, NPAIR, body, 0)
    plsc.subcore_barrier()

    def export_start(off, nr):
        pltpu.async_copy(acc_sh.at[pl.ds(off, nr)],
                         deg_hbm.at[pl.ds(c * N + off, nr)], psem)

    def export_wait(off, nr):
        pltpu.make_async_copy(acc_sh.at[pl.ds(off, nr)],
                              deg_hbm.at[pl.ds(c * N + off, nr)], psem).wait()

    _for_tile_slices(s, export_start, export_wait)


def _sc_agg_body(xs_hbm, cat_hbm, part_hbm,
                 gidx0, gidx1, rows0, rows1, zbuf, acc_sh,
                 gsem0, gsem1, ssem0, ssem1, isem0, isem1, psem):
    c = lax.axis_index("c")
    s = lax.axis_index("s")
    w = c * NS + s

    def fill_zeros(i, carry):
        for j in range(D // 16):
            zbuf[i, pl.ds(j * 16, 16)] = jnp.zeros((16,), jnp.float32)
        return carry

    lax.fori_loop(0, ZCH, fill_zeros, 0)

    def zero_start(off, nr):
        pltpu.async_copy(zbuf.at[pl.ds(0, nr)], acc_sh.at[pl.ds(off, nr)],
                         psem)

    def zero_wait(off, nr):
        pltpu.make_async_copy(zbuf.at[pl.ds(0, nr)],
                              acc_sh.at[pl.ds(off, nr)], psem).wait()

    _for_tile_slices(s, zero_start, zero_wait)
    plsc.subcore_barrier()

    rows_b = (rows0, rows1)
    gsem_b = (gsem0, gsem1)
    ssem_b = (ssem0, ssem1)
    gslot_h = (gidx0, gidx1)
    isem_h = (isem0, isem1)

    def sgather(idx_ref, b):
        pltpu.async_copy(xs_hbm.at[idx_ref], rows_b[b], gsem_b[b])

    def wgather(b):
        pltpu.make_async_copy(xs_hbm.at[gidx0.at[0, 0]], rows_b[b],
                              gsem_b[b]).wait()

    def sscatter(idx_ref, b):
        pltpu.async_copy(rows_b[b], acc_sh.at[idx_ref], ssem_b[b], add=True)

    def wscatter(b):
        pltpu.make_async_copy(rows_b[b], acc_sh.at[gidx0.at[0, 1]],
                              ssem_b[b]).wait()

    def load_group(g, h, sem):
        pltpu.async_copy(cat_hbm.at[w, pl.ds(g * G, G)], gslot_h[h], sem)

    def wait_group(g, h, sem):
        pltpu.make_async_copy(cat_hbm.at[w, pl.ds(g * G, G)], gslot_h[h],
                              sem).wait()

    pltpu.sync_copy(cat_hbm.at[w, pl.ds(0, G)], gidx0)
    sgather(gidx0.at[0, 0], 0)

    def body(p, carry):
        for half in range(2):
            gslot = gslot_h[half]
            nslot = gslot_h[1 - half]

            for k_in in range(G):
                b = k_in % 2
                if half == 0 and k_in == 0:
                    @pl.when(p > 0)
                    def _():
                        wscatter(1 - b)
                else:
                    wscatter(1 - b)
                if k_in < G - 1:
                    sgather(gslot.at[k_in + 1, 0], 1 - b)
                else:
                    if half == 0:
                        wait_group(2 * p + 1, 1 - half, isem_h[1 - half])
                        sgather(nslot.at[0, 0], 1 - b)
                    else:
                        @pl.when(p < NGP - 1)
                        def _():
                            wait_group(2 * p + 2, 0, isem_h[0])
                            sgather(nslot.at[0, 0], 1 - b)
                wgather(b)
                sscatter(gslot.at[k_in, 1], b)
                if k_in == 0:
                    if half == 0:
                        load_group(2 * p + 1, 1, isem_h[1])
                    else:
                        @pl.when(p < NGP - 1)
                        def _():
                            load_group(2 * p + 2, 0, isem_h[0])
        return carry

    lax.fori_loop(0, NGP, body, 0)
    wscatter((NCHUNK - 1) % 2)
    plsc.subcore_barrier()

    def export_start(off, nr):
        pltpu.async_copy(acc_sh.at[pl.ds(off, nr)],
                         part_hbm.at[pl.ds(c * N + off, nr)], psem)

    def export_wait(off, nr):
        pltpu.make_async_copy(acc_sh.at[pl.ds(off, nr)],
                              part_hbm.at[pl.ds(c * N + off, nr)], psem).wait()

    _for_tile_slices(s, export_start, export_wait)


@functools.cache
def _build_sc_kernels():
    mesh = plsc.VectorSubcoreMesh(core_axis_name="c", subcore_axis_name="s",
                                  num_cores=NC, num_subcores=NS)
    sc_degree = pl.kernel(
        _sc_degree_body,
        out_type=jax.ShapeDtypeStruct((NC * N, D), jnp.float32),
        mesh=mesh,
        scratch_types=[
            pltpu.VMEM((2, CHUNK), jnp.int32),
            pltpu.VMEM((2, CHUNK), jnp.int32),
            pltpu.VMEM((CHUNK, D), jnp.float32),
            pltpu.VMEM((CHUNK, D), jnp.float32),
            pltpu.VMEM((ZCH, D), jnp.float32),
            pltpu.VMEM_SHARED((N, D), jnp.float32),
            pltpu.SemaphoreType.DMA,
            pltpu.SemaphoreType.DMA,
            pltpu.SemaphoreType.DMA,
            pltpu.SemaphoreType.DMA,
            pltpu.SemaphoreType.DMA,
        ],
    )
    sc_agg = pl.kernel(
        _sc_agg_body,
        out_type=jax.ShapeDtypeStruct((NC * N, D), jnp.float32),
        mesh=mesh,
        scratch_types=[
            pltpu.VMEM((G, 2, CHUNK), jnp.int32),
            pltpu.VMEM((G, 2, CHUNK), jnp.int32),
            pltpu.VMEM((CHUNK, D), jnp.float32),
            pltpu.VMEM((CHUNK, D), jnp.float32),
            pltpu.VMEM((ZCH, D), jnp.float32),
            pltpu.VMEM_SHARED((N, D), jnp.float32),
            pltpu.SemaphoreType.DMA,
            pltpu.SemaphoreType.DMA,
            pltpu.SemaphoreType.DMA,
            pltpu.SemaphoreType.DMA,
            pltpu.SemaphoreType.DMA,
            pltpu.SemaphoreType.DMA,
            pltpu.SemaphoreType.DMA,
        ],
    )
    return sc_degree, sc_agg


_BLK = 1000
_G = N // _BLK


def _tc_prep_body(h_ref, d0, d1, invo_ref, invi_ref, xs_ref):
    dsum = d0[...] + d1[...]
    deg_o = dsum[:, 0:1]
    deg_i = dsum[:, 16:17]
    invo = lax.rsqrt(jnp.maximum(deg_o, 1.0))
    invi = lax.rsqrt(jnp.maximum(deg_i, 1.0))
    invo_ref[...] = invo
    invi_ref[...] = invi
    xs_ref[...] = h_ref[...] * invo


def _tc_prep(h, deg):
    return pl.pallas_call(
        _tc_prep_body,
        grid=(_G,),
        in_specs=[
            pl.BlockSpec((_BLK, D), lambda i: (i, 0)),
            pl.BlockSpec((_BLK, D), lambda i: (i, 0)),
            pl.BlockSpec((_BLK, D), lambda i: (i + _G, 0)),
        ],
        out_specs=[
            pl.BlockSpec((_BLK, 1), lambda i: (i, 0)),
            pl.BlockSpec((_BLK, 1), lambda i: (i, 0)),
            pl.BlockSpec((_BLK, D), lambda i: (i, 0)),
        ],
        out_shape=[
            jax.ShapeDtypeStruct((N, 1), jnp.float32),
            jax.ShapeDtypeStruct((N, 1), jnp.float32),
            jax.ShapeDtypeStruct((N, D), jnp.float32),
        ],
    )(h, deg, deg)


def _tc_layer_body(p0, p1, invi, invo, w_ref, b_ref, out_ref):
    m = (p0[...] + p1[...]) * invi[...]
    y = jnp.dot(m, w_ref[...], preferred_element_type=jnp.float32) + b_ref[...]
    out_ref[...] = jnp.maximum(y, 0.0) * invo[...]


def _tc_layer(part, invi, invo, w, b):
    return pl.pallas_call(
        _tc_layer_body,
        grid=(_G,),
        in_specs=[
            pl.BlockSpec((_BLK, D), lambda i: (i, 0)),
            pl.BlockSpec((_BLK, D), lambda i: (i + _G, 0)),
            pl.BlockSpec((_BLK, 1), lambda i: (i, 0)),
            pl.BlockSpec((_BLK, 1), lambda i: (i, 0)),
            pl.BlockSpec((D, D), lambda i: (0, 0)),
            pl.BlockSpec((1, D), lambda i: (0, 0)),
        ],
        out_specs=pl.BlockSpec((_BLK, D), lambda i: (i, 0)),
        out_shape=jax.ShapeDtypeStruct((N, D), jnp.float32),
    )(part, part, invi, invo, w, b)


def _tc_final_body(p0, p1, invi, w4, b4, wl1, bl1, wl2, bl2, wo, bo,
                   out_ref, acc):
    i = pl.program_id(0)

    @pl.when(i == 0)
    def _():
        acc[...] = jnp.zeros_like(acc)

    m = (p0[...] + p1[...]) * invi[...]
    y = jnp.dot(m, w4[...], preferred_element_type=jnp.float32) + b4[...]
    y = jnp.maximum(y, 0.0)
    acc[...] += jnp.sum(y, axis=0, keepdims=True)

    @pl.when(i == _G - 1)
    def _():
        hg = acc[...] * (1.0 / N)
        hg = jnp.dot(hg, wl1[...], preferred_element_type=jnp.float32) + bl1[...]
        hg = jnp.maximum(hg, 0.0)
        hg = jnp.dot(hg, wl2[...], preferred_element_type=jnp.float32) + bl2[...]
        hg = jnp.maximum(hg, 0.0)
        out_ref[...] = jnp.dot(hg, wo[...], preferred_element_type=jnp.float32) + bo[...]


def _tc_final(part, invi, w4, b4, wl1, bl1, wl2, bl2, wo, bo):
    return pl.pallas_call(
        _tc_final_body,
        grid=(_G,),
        in_specs=[
            pl.BlockSpec((_BLK, D), lambda i: (i, 0)),
            pl.BlockSpec((_BLK, D), lambda i: (i + _G, 0)),
            pl.BlockSpec((_BLK, 1), lambda i: (i, 0)),
            pl.BlockSpec((D, D), lambda i: (0, 0)),
            pl.BlockSpec((1, D), lambda i: (0, 0)),
            pl.BlockSpec((D, D), lambda i: (0, 0)),
            pl.BlockSpec((1, D), lambda i: (0, 0)),
            pl.BlockSpec((D, D), lambda i: (0, 0)),
            pl.BlockSpec((1, D), lambda i: (0, 0)),
            pl.BlockSpec((D, 1), lambda i: (0, 0)),
            pl.BlockSpec((1, 1), lambda i: (0, 0)),
        ],
        out_specs=pl.BlockSpec((1, 1), lambda i: (0, 0)),
        out_shape=jax.ShapeDtypeStruct((1, 1), jnp.float32),
        scratch_shapes=[pltpu.VMEM((1, D), jnp.float32)],
    )(part, part, invi, w4, b4, wl1, bl1, wl2, bl2, wo, bo)


def kernel(h, edge_index, W1, b1, W2, b2, W3, b3, W4, b4,
           Wl1, bl1, Wl2, bl2, Wo, bo):
    cat = jnp.transpose(edge_index.reshape(2, NW, NCHUNK, CHUNK), (1, 2, 0, 3))

    _sc_degree, _sc_agg = _build_sc_kernels()
    deg = _sc_degree(cat)
    invo, invi, xs = _tc_prep(h, deg)

    for w, b in ((W1, b1), (W2, b2), (W3, b3)):
        part = _sc_agg(xs, cat)
        xs = _tc_layer(part, invi, invo, w, b.reshape(1, D))

    part = _sc_agg(xs, cat)
    return _tc_final(part, invi, W4, b4.reshape(1, D),
                     Wl1, bl1.reshape(1, D), Wl2, bl2.reshape(1, D),
                     Wo, bo.reshape(1, 1))
